# baseline (speedup 1.0000x reference)
import contextlib
import functools

import numpy as np
import jax
import jax.numpy as jnp
from jax import lax
from jax.experimental import pallas as pl
from jax.experimental.pallas import tpu as pltpu
from jax.experimental.pallas import tpu_sc as plsc

_NEG = 16
_L = 16
_NC, _NS = 2, 16
_NW = _NC * _NS
_D = 128
_DC = _D // _L
_N = 8192
_B = _N // 2
_PB = _B // _NW
_GA = 16
_NG = _PB // _GA
_CHUNK = 128
_PITCH = 17

_LN2 = 0.6931471805599453


@functools.cache
def _neg_inds_const(b: int) -> np.ndarray:
    def build():
        rows = jnp.arange(b)
        logw = jnp.zeros((b, 2 * b), dtype=jnp.float32)
        logw = logw.at[rows, rows].set(-jnp.inf)
        logw = logw.at[rows, rows + b].set(-jnp.inf)
        g = jax.random.gumbel(jax.random.key(42), (b, 2 * b), dtype=jnp.float32)
        _, neg = lax.top_k(logw + g, _NEG)
        return neg
    with jax.set_mesh(None):
        try:
            cpu = jax.devices("cpu")[0]
            ctx = jax.default_device(cpu)
        except Exception:
            ctx = contextlib.nullcontext()
        with ctx:
            neg = build()
    return np.asarray(neg, dtype=np.int32)


def _vlog(x):
    bits = lax.bitcast_convert_type(x, jnp.int32)
    e = lax.shift_right_logical(bits, 23) - 127
    m_bits = (bits & 0x7FFFFF) | 0x3F800000
    m = lax.bitcast_convert_type(m_bits, jnp.float32)
    big = m > 1.4142135623730951
    m = jnp.where(big, m * 0.5, m)
    e = e + jnp.where(big, 1, 0)
    s = (m - 1.0) / (m + 1.0)
    z = s * s
    p = z * (1.0 / 9.0) + (1.0 / 7.0)
    p = z * p + (1.0 / 5.0)
    p = z * p + (1.0 / 3.0)
    p = z * p + 1.0
    return e.astype(jnp.float32) * _LN2 + (2.0 * s) * p


def _loss_terms(dists, positive):
    probit = 1.0 / (1.0 + dists)
    if positive:
        val = probit
    else:
        val = 1.0 - probit
    val = jnp.minimum(jnp.maximum(val, 0.0001), 1.0)
    return -_vlog(val)


def _make_norm_call():
    def norm_kernel(f_ref, o_ref):
        x = f_ref[...]
        o_ref[...] = jnp.sum(x * x, axis=1)
    return pl.pallas_call(
        norm_kernel,
        out_shape=jax.ShapeDtypeStruct((_N,), jnp.float32))


def _make_sc_call():
    mesh = plsc.VectorSubcoreMesh(
        core_axis_name="c", subcore_axis_name="s",
        num_cores=_NC, num_subcores=_NS)

    @functools.partial(
        pl.kernel,
        out_type=jax.ShapeDtypeStruct((_NW, _L), jnp.float32),
        mesh=mesh,
        compiler_params=pltpu.CompilerParams(needs_layout_passes=False),
        scratch_types=[
            pltpu.VMEM((_PB, _D), jnp.float32),
            pltpu.VMEM((_PB, _D), jnp.float32),
            pltpu.VMEM((_NEG, _CHUNK), jnp.int32),
            pltpu.VMEM((2, _GA * _NEG, _D), jnp.float32),
            pltpu.VMEM((_NEG * _PITCH,), jnp.float32),
            pltpu.VMEM((_GA * _PITCH,), jnp.float32),
            pltpu.VMEM((_L,), jnp.float32),
            pltpu.VMEM((_N,), jnp.float32),
            pltpu.SemaphoreType.DMA,
            pltpu.SemaphoreType.DMA,
            pltpu.SemaphoreType.DMA,
            pltpu.SemaphoreType.DMA,
        ],
    )
    def sc_loss(feat_hbm, idx_hbm, norm_hbm, out_hbm,
                origs_v, pos_v, idx_v, nbr_v, nscr, pscr, loss_v, norm_v,
                sem0, sem1, semp, semn):
        w = lax.axis_index("s") * _NC + lax.axis_index("c")
        ab = w * _PB
        pltpu.sync_copy(idx_hbm.at[pl.ds(w * _NEG, _NEG)], idx_v)

        sems = (sem0, sem1)

        def start_group(g, slot):
            pltpu.async_copy(
                feat_hbm.at[idx_v.at[2 * g]],
                nbr_v.at[slot, pl.ds(0, _CHUNK)], sems[slot])
            pltpu.async_copy(
                feat_hbm.at[idx_v.at[2 * g + 1]],
                nbr_v.at[slot, pl.ds(_CHUNK, _CHUNK)], sems[slot])

        def drain(slot):
            pltpu.make_async_copy(
                feat_hbm.at[pl.ds(0, _GA * _NEG)],
                nbr_v.at[slot], sems[slot]).wait()

        start_group(0, 0)
        start_group(1, 1)
        cpo = pltpu.async_copy(feat_hbm.at[pl.ds(ab, _PB)], origs_v, semp)
        cpp = pltpu.async_copy(feat_hbm.at[pl.ds(_B + ab, _PB)], pos_v, semp)
        cpn = pltpu.async_copy(norm_hbm, norm_v, semn)
        cpo.wait()
        cpp.wait()
        cpn.wait()

        lanes = lax.iota(jnp.int32, _L)
        lanes_p = lanes * _PITCH

        def compute_group(g, slot, acc0):
            def anchor_step(al, acc):
                a = g * _GA + al
                o = [origs_v[a, pl.ds(k * _L, _L)] for k in range(_DC)]
                pv = [pos_v[a, pl.ds(k * _L, _L)] for k in range(_DC)]
                pacc = None
                for k in range(_DC):
                    dd = o[k] - pv[k]
                    dd = dd * dd
                    pacc = dd if pacc is None else pacc + dd
                plsc.store_scatter(pscr, [lanes + al * _PITCH], pacc)
                naccs = [None] * _NEG
                for k in range(_DC):
                    ok = o[k]
                    for j in range(_NEG):
                        r = al * _NEG + j
                        dd = ok * nbr_v[slot, r, pl.ds(k * _L, _L)]
                        naccs[j] = dd if naccs[j] is None else naccs[j] + dd
                for j in range(_NEG):
                    plsc.store_scatter(nscr, [lanes + j * _PITCH], naccs[j])
                dn = plsc.load_gather(nscr, [lanes_p])
                for c in range(1, _L):
                    dn = dn + plsc.load_gather(nscr, [lanes_p + c])
                iv = idx_v[2 * g + al // 8, pl.ds((al % 8) * _L, _L)]
                nn = plsc.load_gather(norm_v, [iv])
                on = plsc.load_gather(norm_v, [lanes * 0 + (ab + a)])
                d = (on + nn) - (dn + dn)
                return acc + _loss_terms(d, positive=False)

            acc1 = lax.fori_loop(0, _GA, anchor_step, acc0)
            dp = plsc.load_gather(pscr, [lanes_p])
            for c in range(1, _L):
                dp = dp + plsc.load_gather(pscr, [lanes_p + c])
            return acc1 + _loss_terms(dp, positive=True)

        def super_step(h, loss_acc):
            for sl in range(2):
                g = 2 * h + sl
                drain(sl)
                loss_acc = compute_group(g, sl, loss_acc)
                start_group((g + 2) & (_NG - 1), sl)
            return loss_acc

        loss_acc = lax.fori_loop(0, _NG // 2, super_step,
                                 jnp.zeros((_L,), jnp.float32))
        drain(0)
        drain(1)

        loss_v[...] = loss_acc
        pltpu.sync_copy(loss_v, out_hbm.at[w])

    return sc_loss


_sc_call = None
_NEG_TABLE = _neg_inds_const(_B).reshape(_NW * _NEG, _CHUNK)


_norm_call = None


def kernel(features):
    global _sc_call, _norm_call
    n, d = features.shape
    assert (n, d) == (_N, _D)
    idx = jnp.asarray(_NEG_TABLE)
    if _sc_call is None:
        _sc_call = _make_sc_call()
        _norm_call = _make_norm_call()
    norms = _norm_call(features)
    partial = _sc_call(features, idx, norms)
    return jnp.sum(partial) / np.float32(_B * (_NEG + 1))


# speedup vs baseline: 1.1476x; 1.1476x over previous
import contextlib
import functools

import numpy as np
import jax
import jax.numpy as jnp
from jax import lax
from jax.experimental import pallas as pl
from jax.experimental.pallas import tpu as pltpu
from jax.experimental.pallas import tpu_sc as plsc

_NEG = 16
_L = 16
_NC, _NS = 2, 16
_NW = _NC * _NS
_D = 128
_DC = _D // _L
_N = 8192
_B = _N // 2
_PB = _B // _NW
_GA = 16
_NG = _PB // _GA
_CHUNK = 128
_PITCH = 17

_LN2 = 0.6931471805599453


@functools.cache
def _neg_inds_const(b: int) -> np.ndarray:
    def build():
        rows = jnp.arange(b)
        logw = jnp.zeros((b, 2 * b), dtype=jnp.float32)
        logw = logw.at[rows, rows].set(-jnp.inf)
        logw = logw.at[rows, rows + b].set(-jnp.inf)
        g = jax.random.gumbel(jax.random.key(42), (b, 2 * b), dtype=jnp.float32)
        _, neg = lax.top_k(logw + g, _NEG)
        return neg
    with jax.set_mesh(None):
        try:
            cpu = jax.devices("cpu")[0]
            ctx = jax.default_device(cpu)
        except Exception:
            ctx = contextlib.nullcontext()
        with ctx:
            neg = build()
    return np.asarray(neg, dtype=np.int32)


def _vlog(x):
    bits = lax.bitcast_convert_type(x, jnp.int32)
    e = lax.shift_right_logical(bits, 23) - 127
    m_bits = (bits & 0x7FFFFF) | 0x3F800000
    m = lax.bitcast_convert_type(m_bits, jnp.float32)
    big = m > 1.4142135623730951
    m = jnp.where(big, m * 0.5, m)
    e = e + jnp.where(big, 1, 0)
    s = (m - 1.0) / (m + 1.0)
    z = s * s
    p = z * (1.0 / 9.0) + (1.0 / 7.0)
    p = z * p + (1.0 / 5.0)
    p = z * p + (1.0 / 3.0)
    p = z * p + 1.0
    return e.astype(jnp.float32) * _LN2 + (2.0 * s) * p


def _loss_terms(dists, positive):
    probit = 1.0 / (1.0 + dists)
    if positive:
        val = probit
    else:
        val = 1.0 - probit
    val = jnp.minimum(jnp.maximum(val, 0.0001), 1.0)
    return -_vlog(val)


def _make_sc_call():
    mesh = plsc.VectorSubcoreMesh(
        core_axis_name="c", subcore_axis_name="s",
        num_cores=_NC, num_subcores=_NS)

    @functools.partial(
        pl.kernel,
        out_type=jax.ShapeDtypeStruct((_NW, _L), jnp.float32),
        mesh=mesh,
        compiler_params=pltpu.CompilerParams(needs_layout_passes=False),
        scratch_types=[
            pltpu.VMEM((_PB, _D), jnp.float32),
            pltpu.VMEM((_PB, _D), jnp.float32),
            pltpu.VMEM((_NEG, _CHUNK), jnp.int32),
            pltpu.VMEM((2, _GA * _NEG, _D), jnp.float32),
            pltpu.VMEM((_NEG * _PITCH,), jnp.float32),
            pltpu.VMEM((_GA * _PITCH,), jnp.float32),
            pltpu.VMEM((_L,), jnp.float32),
            pltpu.SemaphoreType.DMA,
            pltpu.SemaphoreType.DMA,
            pltpu.SemaphoreType.DMA,
        ],
    )
    def sc_loss(feat_hbm, idx_hbm, out_hbm,
                origs_v, pos_v, idx_v, nbr_v, nscr, pscr, loss_v,
                sem0, sem1, semp):
        w = lax.axis_index("s") * _NC + lax.axis_index("c")
        ab = w * _PB
        pltpu.sync_copy(idx_hbm.at[pl.ds(w * _NEG, _NEG)], idx_v)

        sems = (sem0, sem1)

        def start_group(g, slot):
            pltpu.async_copy(
                feat_hbm.at[idx_v.at[2 * g]],
                nbr_v.at[slot, pl.ds(0, _CHUNK)], sems[slot])
            pltpu.async_copy(
                feat_hbm.at[idx_v.at[2 * g + 1]],
                nbr_v.at[slot, pl.ds(_CHUNK, _CHUNK)], sems[slot])

        def drain(slot):
            pltpu.make_async_copy(
                feat_hbm.at[pl.ds(0, _GA * _NEG)],
                nbr_v.at[slot], sems[slot]).wait()

        start_group(0, 0)
        start_group(1, 1)
        cpo = pltpu.async_copy(feat_hbm.at[pl.ds(ab, _PB)], origs_v, semp)
        cpp = pltpu.async_copy(feat_hbm.at[pl.ds(_B + ab, _PB)], pos_v, semp)
        cpo.wait()
        cpp.wait()

        lanes = lax.iota(jnp.int32, _L)
        lanes_p = lanes * _PITCH

        def compute_group(g, slot, acc0):
            def anchor_step(al, acc):
                a = g * _GA + al
                o = [origs_v[a, pl.ds(k * _L, _L)] for k in range(_DC)]
                pv = [pos_v[a, pl.ds(k * _L, _L)] for k in range(_DC)]
                pacc = None
                naccs = [None] * _NEG
                for k in range(_DC):
                    ok = o[k]
                    dd = ok - pv[k]
                    dd = dd * dd
                    pacc = dd if pacc is None else pacc + dd
                    for j in range(_NEG):
                        r = al * _NEG + j
                        dd = ok - nbr_v[slot, r, pl.ds(k * _L, _L)]
                        dd = dd * dd
                        naccs[j] = dd if naccs[j] is None else naccs[j] + dd
                plsc.store_scatter(pscr, [lanes + al * _PITCH], pacc)
                for j in range(_NEG):
                    plsc.store_scatter(nscr, [lanes + j * _PITCH], naccs[j])
                dn = plsc.load_gather(nscr, [lanes_p])
                for c in range(1, _L):
                    dn = dn + plsc.load_gather(nscr, [lanes_p + c])
                return acc + _loss_terms(dn, positive=False)

            acc1 = lax.fori_loop(0, _GA, anchor_step, acc0)
            dp = plsc.load_gather(pscr, [lanes_p])
            for c in range(1, _L):
                dp = dp + plsc.load_gather(pscr, [lanes_p + c])
            return acc1 + _loss_terms(dp, positive=True)

        def super_step(h, loss_acc):
            for sl in range(2):
                g = 2 * h + sl
                drain(sl)
                loss_acc = compute_group(g, sl, loss_acc)
                start_group((g + 2) & (_NG - 1), sl)
            return loss_acc

        loss_acc = lax.fori_loop(0, _NG // 2, super_step,
                                 jnp.zeros((_L,), jnp.float32))
        drain(0)
        drain(1)

        loss_v[...] = loss_acc
        pltpu.sync_copy(loss_v, out_hbm.at[w])

    return sc_loss


_sc_call = None
_NEG_TABLE = _neg_inds_const(_B).reshape(_NW * _NEG, _CHUNK)


def kernel(features):
    global _sc_call
    n, d = features.shape
    assert (n, d) == (_N, _D)
    idx = jnp.asarray(_NEG_TABLE)
    if _sc_call is None:
        _sc_call = _make_sc_call()
    partial = _sc_call(features, idx)
    return jnp.sum(partial) / np.float32(_B * (_NEG + 1))
